# baseline (device time: 22911 ns/iter reference)
import jax
import jax.numpy as jnp
from jax import lax
from jax.experimental import pallas as pl
from jax.experimental.pallas import tpu as pltpu

Z_DEV = 4
B, SQ, SKV, H, D = 8, 1, 512, 8, 64
HD = H * D
PACK = HD + 128


def kernel(Q, K, V):
    k2 = K.reshape(B, SKV, HD)
    v2 = jnp.concatenate(
        [V.reshape(B, SKV, HD), jnp.ones((B, SKV, H), jnp.float32)], axis=2
    )
    q2 = Q.reshape(B, HD)
    e2 = (jnp.arange(HD)[:, None] // D == jnp.arange(H)[None, :])
    qblk = q2[:, :, None] * e2[None].astype(jnp.float32)

    def body(qblk_ref, k_ref, v_ref, out_ref, comm, send_sems, recv_sems):
        my_x = lax.axis_index("x")
        my_y = lax.axis_index("y")
        my_z = lax.axis_index("z")

        barrier_sem = pltpu.get_barrier_semaphore()
        for r in (1, 2, 3):
            pl.semaphore_signal(
                barrier_sem,
                inc=1,
                device_id=(my_x, my_y, (my_z + r) % Z_DEV),
                device_id_type=pl.DeviceIdType.MESH,
            )

        ids_hd = lax.broadcasted_iota(jnp.int32, (H, HD), 1) // D
        ids_h = lax.broadcasted_iota(jnp.int32, (H, HD), 0)
        e8 = (ids_hd == ids_h).astype(jnp.float32)
        ca = lax.broadcasted_iota(jnp.int32, (H, HD + H), 1)
        ha = lax.broadcasted_iota(jnp.int32, (H, HD + H), 0)
        sel = jnp.where(ca < HD, ca // D, ca - HD)
        e8a = (sel == ha).astype(jnp.float32)

        scale = D ** -0.5
        rows = []
        for b in range(B):
            s_t = lax.dot_general(
                qblk_ref[b], k_ref[b], (((0,), (1,)), ((), ()))
            )
            p_t = jnp.exp(s_t * scale)
            cross = jax.lax.dot(p_t, v_ref[b])
            rows.append(jnp.sum(cross * e8a, axis=0, keepdims=True))
        part = jnp.concatenate(rows, axis=0)
        comm[0] = jnp.concatenate(
            [part, jnp.zeros((B, PACK - HD - H), jnp.float32)], axis=1
        )

        pl.semaphore_wait(barrier_sem, Z_DEV - 1)

        sends = []
        for r in (1, 2, 3):
            send = pltpu.make_async_remote_copy(
                src_ref=comm.at[0],
                dst_ref=comm.at[Z_DEV - r],
                send_sem=send_sems.at[r - 1],
                recv_sem=recv_sems.at[Z_DEV - r - 1],
                device_id=(my_x, my_y, (my_z + r) % Z_DEV),
                device_id_type=pl.DeviceIdType.MESH,
            )
            send.start()
            sends.append(send)
        for t in (1, 2, 3):
            recv = pltpu.make_async_remote_copy(
                src_ref=comm.at[0],
                dst_ref=comm.at[t],
                send_sem=send_sems.at[t - 1],
                recv_sem=recv_sems.at[t - 1],
                device_id=(my_x, my_y, my_z),
                device_id_type=pl.DeviceIdType.MESH,
            )
            recv.wait_recv()

        tot = jnp.sum(comm[...], axis=0)
        o_sum = tot[:, :HD]
        l_sum = tot[:, HD:HD + H]
        l_flat = jax.lax.dot(l_sum, e8)
        out_ref[...] = o_sum / l_flat

        for send in sends:
            send.wait_send()

    out = pl.pallas_call(
        body,
        out_shape=jax.ShapeDtypeStruct((B, HD), jnp.float32),
        in_specs=[
            pl.BlockSpec(memory_space=pltpu.VMEM),
            pl.BlockSpec(memory_space=pltpu.VMEM),
            pl.BlockSpec(memory_space=pltpu.VMEM),
        ],
        out_specs=pl.BlockSpec(memory_space=pltpu.VMEM),
        scratch_shapes=[
            pltpu.VMEM((Z_DEV, B, PACK), jnp.float32),
            pltpu.SemaphoreType.DMA((Z_DEV - 1,)),
            pltpu.SemaphoreType.DMA((Z_DEV - 1,)),
        ],
        compiler_params=pltpu.CompilerParams(collective_id=0),
    )(qblk, k2, v2)
    return out.reshape(B, SQ, H, D)
